# baseline (speedup 1.0000x reference)
import dataclasses
import functools

import jax
import jax.numpy as jnp
from jax import lax
from jax.experimental import pallas as pl
from jax.experimental.pallas import tpu as pltpu
from jax.experimental.pallas import tpu_sc as plsc

N = 10000
NPAD = 10112
H = 256
HH = H // 2
E = 160000
EPAD = 161280
NS = 16
CHUNK = 72
NCHUNK = EPAD // NS // CHUNK
EPT = EPAD // NS
RPT = NPAD // NS
BLK = 1000
GS = (0, 24, 40, 56, 72)

_mesh = plsc.VectorSubcoreMesh(core_axis_name="c", subcore_axis_name="s",
                               num_cores=2, num_subcores=16)


def _agg_sc(table_l, table_r, src1d, dst1d, zeros_l, with_hist):
    out_type = [
        jax.ShapeDtypeStruct((NPAD, HH), jnp.float32),
        jax.ShapeDtypeStruct((NPAD, HH), jnp.float32),
    ]
    scratch = [
        pltpu.VMEM((EPT,), jnp.int32),
        pltpu.VMEM((EPT,), jnp.int32),
        pltpu.VMEM((CHUNK, HH), jnp.float32),
        pltpu.VMEM((CHUNK, HH), jnp.float32),
        pltpu.VMEM_SHARED((NPAD, HH), jnp.float32),
        [pltpu.SemaphoreType.DMA] * 4,
        [pltpu.SemaphoreType.DMA] * 4,
    ]
    cp = pltpu.CompilerParams()
    if with_hist:
        out_type.append(jax.ShapeDtypeStruct((NS, NPAD), jnp.float32))
        scratch.append(pltpu.VMEM((NPAD,), jnp.float32))
        if "needs_layout_passes" in pltpu.CompilerParams.__dataclass_fields__:
            cp = dataclasses.replace(cp, needs_layout_passes=False)

    @functools.partial(pl.kernel, out_type=tuple(out_type), mesh=_mesh,
                       scratch_types=scratch, compiler_params=cp)
    def agg(tl_hbm, tr_hbm, src_hbm, dst_hbm, z_hbm, outl_hbm, outr_hbm,
            *rest):
        if with_hist:
            (hist_hbm, srcv, dstv, bufa, bufb, acc, sems_a, sems_b,
             hist) = rest
        else:
            srcv, dstv, bufa, bufb, acc, sems_a, sems_b = rest
        cid = lax.axis_index("c")
        sid = lax.axis_index("s")
        rbase = pl.multiple_of(sid * RPT, 8)
        fbase = pl.multiple_of(sid * EPT, 8)
        pltpu.sync_copy(z_hbm.at[pl.ds(rbase, RPT)], acc.at[pl.ds(rbase, RPT)])
        pltpu.sync_copy(src_hbm.at[pl.ds(fbase, EPT)], srcv)
        pltpu.sync_copy(dst_hbm.at[pl.ds(fbase, EPT)], dstv)
        plsc.subcore_barrier()

        if with_hist:
            @pl.when(cid == 0)
            def _():
                @pl.loop(0, NPAD, step=16)
                def _(o):
                    hist[pl.ds(o, 16)] = jnp.zeros((16,), jnp.float32)

                @pl.loop(0, EPT, step=16)
                def _(o):
                    idx = dstv[pl.ds(o, 16)]
                    c, last = plsc.scan_count(idx)
                    plsc.addupdate_scatter(hist, [idx],
                                           c.astype(jnp.float32), mask=last)

                pltpu.sync_copy(hist, hist_hbm.at[sid])

        def dst_at(i):
            return dstv.at[pl.ds(pl.multiple_of(i * CHUNK, 8), CHUNK)]

        def edge_loop(table):
            def sub(i, buf, sems, k):
                lo, hi = GS[k], GS[k + 1]
                idx = srcv.at[pl.ds(pl.multiple_of(i * CHUNK + lo, 8), hi - lo)]
                return table.at[idx], buf.at[pl.ds(lo, hi - lo)], sems[k]

            def start_g(i, buf, sems):
                for k in range(4):
                    pltpu.async_copy(*sub(i, buf, sems, k))

            def wait_g(i, buf, sems):
                for k in range(4):
                    pltpu.make_async_copy(*sub(i, buf, sems, k)).wait()

            start_g(0, bufa, sems_a)

            @pl.loop(0, NCHUNK, step=2)
            def _(i):
                wait_g(i, bufa, sems_a)
                start_g(i + 1, bufb, sems_b)
                pltpu.sync_copy(bufa, acc.at[dst_at(i)], add=True)
                wait_g(i + 1, bufb, sems_b)

                @pl.when(i + 2 < NCHUNK)
                def _():
                    start_g(i + 2, bufa, sems_a)

                pltpu.sync_copy(bufb, acc.at[dst_at(i + 1)], add=True)

        @pl.when(cid == 0)
        def _():
            edge_loop(tl_hbm)

        @pl.when(cid == 1)
        def _():
            edge_loop(tr_hbm)

        plsc.subcore_barrier()

        @pl.when(cid == 0)
        def _():
            pltpu.sync_copy(acc.at[pl.ds(rbase, RPT)], outl_hbm.at[pl.ds(rbase, RPT)])

        @pl.when(cid == 1)
        def _():
            pltpu.sync_copy(acc.at[pl.ds(rbase, RPT)], outr_hbm.at[pl.ds(rbase, RPT)])

    return agg(table_l, table_r, src1d, dst1d, zeros_l)


def _dense_tc(sum_l, sum_r, cnt, x_l, x_r, w_l, b_l, w_r, relu, half_out):
    def body(sl_ref, sr_ref, c_ref, xl_ref, xr_ref, wl_ref, b_ref, wr_ref, *outs):
        inv = 1.0 / jnp.maximum(c_ref[...], 1.0)
        acc = jnp.dot(sl_ref[...] * inv, wl_ref[0:HH, :],
                      preferred_element_type=jnp.float32)
        acc = acc + jnp.dot(sr_ref[...] * inv, wl_ref[HH:H, :],
                            preferred_element_type=jnp.float32)
        acc = acc + jnp.dot(xl_ref[...], wr_ref[0:HH, :],
                            preferred_element_type=jnp.float32)
        acc = acc + jnp.dot(xr_ref[...], wr_ref[HH:H, :],
                            preferred_element_type=jnp.float32)
        acc = acc + b_ref[...]
        if relu:
            acc = jnp.maximum(acc, 0.0)
        if half_out:
            outs[0][...] = acc[:, 0:HH]
            outs[1][...] = acc[:, HH:H]
        else:
            outs[0][...] = acc

    if half_out:
        out_shape = (jax.ShapeDtypeStruct((N, HH), jnp.float32),
                     jax.ShapeDtypeStruct((N, HH), jnp.float32))
        out_specs = (pl.BlockSpec((BLK, HH), lambda i: (i, 0)),
                     pl.BlockSpec((BLK, HH), lambda i: (i, 0)))
    else:
        out_shape = jax.ShapeDtypeStruct((N, H), jnp.float32)
        out_specs = pl.BlockSpec((BLK, H), lambda i: (i, 0))

    return pl.pallas_call(
        body,
        grid=(N // BLK,),
        in_specs=[
            pl.BlockSpec((BLK, HH), lambda i: (i, 0)),
            pl.BlockSpec((BLK, HH), lambda i: (i, 0)),
            pl.BlockSpec((BLK, 1), lambda i: (i, 0)),
            pl.BlockSpec((BLK, HH), lambda i: (i, 0)),
            pl.BlockSpec((BLK, HH), lambda i: (i, 0)),
            pl.BlockSpec((H, H), lambda i: (0, 0)),
            pl.BlockSpec((1, H), lambda i: (0, 0)),
            pl.BlockSpec((H, H), lambda i: (0, 0)),
        ],
        out_specs=out_specs,
        out_shape=out_shape,
    )(sum_l, sum_r, cnt, x_l, x_r, w_l, b_l.reshape(1, H), w_r)


def kernel(ei_g2go, ei_go2g, gene_emb, go_emb,
           W1l_g2go, b1_g2go, W1r_g2go, W1l_go2g, b1_go2g, W1r_go2g,
           W2l_g2go, b2_g2go, W2r_g2go, W2l_go2g, b2_go2g, W2r_go2g):
    pad = jnp.arange(EPAD - E, dtype=jnp.int32)
    pad_src = pad % N
    pad_dst = N + pad % (NPAD - N)
    src_a = jnp.concatenate([ei_g2go[0], pad_src])
    dst_a = jnp.concatenate([ei_g2go[1], pad_dst])
    src_b = jnp.concatenate([ei_go2g[0], pad_src])
    dst_b = jnp.concatenate([ei_go2g[1], pad_dst])
    zeros_l = jnp.zeros((NPAD, HH), jnp.float32)
    gel, ger = gene_emb[:, :HH], gene_emb[:, HH:]
    gol, gor = go_emb[:, :HH], go_emb[:, HH:]

    s1go_l, s1go_r, hist_a = _agg_sc(gel, ger, src_a, dst_a, zeros_l, True)
    s1ge_l, s1ge_r, hist_b = _agg_sc(gol, gor, src_b, dst_b, zeros_l, True)
    cnt_go = jnp.sum(hist_a, axis=0).reshape(NPAD, 1)
    cnt_gene = jnp.sum(hist_b, axis=0).reshape(NPAD, 1)

    go1l, go1r = _dense_tc(s1go_l, s1go_r, cnt_go, gol, gor,
                           W1l_g2go, b1_g2go, W1r_g2go, True, True)
    ge1l, ge1r = _dense_tc(s1ge_l, s1ge_r, cnt_gene, gel, ger,
                           W1l_go2g, b1_go2g, W1r_go2g, True, True)

    s2go_l, s2go_r = _agg_sc(ge1l, ge1r, src_a, dst_a, zeros_l, False)
    s2ge_l, s2ge_r = _agg_sc(go1l, go1r, src_b, dst_b, zeros_l, False)

    go2 = _dense_tc(s2go_l, s2go_r, cnt_go, go1l, go1r,
                    W2l_g2go, b2_g2go, W2r_g2go, False, False)
    gene2 = _dense_tc(s2ge_l, s2ge_r, cnt_gene, ge1l, ge1r,
                      W2l_go2g, b2_go2g, W2r_go2g, False, False)
    return (gene2, go2)


# speedup vs baseline: 1.0109x; 1.0109x over previous
import dataclasses
import functools

import jax
import jax.numpy as jnp
from jax import lax
from jax.experimental import pallas as pl
from jax.experimental.pallas import tpu as pltpu
from jax.experimental.pallas import tpu_sc as plsc

N = 10000
NPAD = 10112
H = 256
HH = H // 2
E = 160000
EPAD = 161280
NS = 16
CHUNK = 72
NCHUNK = EPAD // NS // CHUNK
EPT = EPAD // NS
RPT = NPAD // NS
BLK = 1000
GS = (0, 24, 40, 56, 72)

_mesh = plsc.VectorSubcoreMesh(core_axis_name="c", subcore_axis_name="s",
                               num_cores=2, num_subcores=16)


def _agg_sc(table_l, table_r, src1d, dst1d, zeros_l, with_hist):
    out_type = [
        jax.ShapeDtypeStruct((NPAD, HH), jnp.float32),
        jax.ShapeDtypeStruct((NPAD, HH), jnp.float32),
    ]
    scratch = [
        pltpu.VMEM((EPT,), jnp.int32),
        pltpu.VMEM((EPT,), jnp.int32),
        pltpu.VMEM((CHUNK, HH), jnp.float32),
        pltpu.VMEM((CHUNK, HH), jnp.float32),
        pltpu.VMEM_SHARED((NPAD, HH), jnp.float32),
        [pltpu.SemaphoreType.DMA] * 4,
        [pltpu.SemaphoreType.DMA] * 4,
    ]
    cp = pltpu.CompilerParams()
    if with_hist:
        out_type.append(jax.ShapeDtypeStruct((NS, NPAD), jnp.float32))
        scratch.append(pltpu.VMEM((NPAD,), jnp.float32))
        if "needs_layout_passes" in pltpu.CompilerParams.__dataclass_fields__:
            cp = dataclasses.replace(cp, needs_layout_passes=False)

    @functools.partial(pl.kernel, out_type=tuple(out_type), mesh=_mesh,
                       scratch_types=scratch, compiler_params=cp)
    def agg(tl_hbm, tr_hbm, src_hbm, dst_hbm, z_hbm, outl_hbm, outr_hbm,
            *rest):
        if with_hist:
            (hist_hbm, srcv, dstv, bufa, bufb, acc, sems_a, sems_b,
             hist) = rest
        else:
            srcv, dstv, bufa, bufb, acc, sems_a, sems_b = rest
        cid = lax.axis_index("c")
        sid = lax.axis_index("s")
        rbase = pl.multiple_of(sid * RPT, 8)
        fbase = pl.multiple_of(sid * EPT, 8)
        pltpu.async_copy(z_hbm.at[pl.ds(rbase, RPT)], acc.at[pl.ds(rbase, RPT)], sems_a[0])
        pltpu.async_copy(src_hbm.at[pl.ds(fbase, EPT)], srcv, sems_a[1])
        pltpu.async_copy(dst_hbm.at[pl.ds(fbase, EPT)], dstv, sems_a[2])
        pltpu.make_async_copy(z_hbm.at[pl.ds(rbase, RPT)], acc.at[pl.ds(rbase, RPT)], sems_a[0]).wait()
        pltpu.make_async_copy(src_hbm.at[pl.ds(fbase, EPT)], srcv, sems_a[1]).wait()
        pltpu.make_async_copy(dst_hbm.at[pl.ds(fbase, EPT)], dstv, sems_a[2]).wait()
        plsc.subcore_barrier()

        if with_hist:
            @pl.when(cid == 0)
            def _():
                @pl.loop(0, NPAD, step=16)
                def _(o):
                    hist[pl.ds(o, 16)] = jnp.zeros((16,), jnp.float32)

                @pl.loop(0, EPT, step=16)
                def _(o):
                    idx = dstv[pl.ds(o, 16)]
                    c, last = plsc.scan_count(idx)
                    plsc.addupdate_scatter(hist, [idx],
                                           c.astype(jnp.float32), mask=last)

                pltpu.sync_copy(hist, hist_hbm.at[sid])

        def dst_at(i):
            return dstv.at[pl.ds(pl.multiple_of(i * CHUNK, 8), CHUNK)]

        def edge_loop(table):
            def sub(i, buf, sems, k):
                lo, hi = GS[k], GS[k + 1]
                idx = srcv.at[pl.ds(pl.multiple_of(i * CHUNK + lo, 8), hi - lo)]
                return table.at[idx], buf.at[pl.ds(lo, hi - lo)], sems[k]

            def start_g(i, buf, sems):
                for k in range(4):
                    pltpu.async_copy(*sub(i, buf, sems, k))

            def wait_g(i, buf, sems):
                for k in range(4):
                    pltpu.make_async_copy(*sub(i, buf, sems, k)).wait()

            start_g(0, bufa, sems_a)

            @pl.loop(0, NCHUNK, step=2)
            def _(i):
                wait_g(i, bufa, sems_a)
                start_g(i + 1, bufb, sems_b)
                pltpu.sync_copy(bufa, acc.at[dst_at(i)], add=True)
                wait_g(i + 1, bufb, sems_b)

                @pl.when(i + 2 < NCHUNK)
                def _():
                    start_g(i + 2, bufa, sems_a)

                pltpu.sync_copy(bufb, acc.at[dst_at(i + 1)], add=True)

        @pl.when(cid == 0)
        def _():
            edge_loop(tl_hbm)

        @pl.when(cid == 1)
        def _():
            edge_loop(tr_hbm)

        plsc.subcore_barrier()

        @pl.when(cid == 0)
        def _():
            pltpu.sync_copy(acc.at[pl.ds(rbase, RPT)], outl_hbm.at[pl.ds(rbase, RPT)])

        @pl.when(cid == 1)
        def _():
            pltpu.sync_copy(acc.at[pl.ds(rbase, RPT)], outr_hbm.at[pl.ds(rbase, RPT)])

    return agg(table_l, table_r, src1d, dst1d, zeros_l)


def _dense_tc(sum_l, sum_r, cnt, x_l, x_r, w_l, b_l, w_r, relu, half_out):
    def body(sl_ref, sr_ref, c_ref, xl_ref, xr_ref, wl_ref, b_ref, wr_ref, *outs):
        inv = 1.0 / jnp.maximum(c_ref[...], 1.0)
        acc = jnp.dot(sl_ref[...] * inv, wl_ref[0:HH, :],
                      preferred_element_type=jnp.float32)
        acc = acc + jnp.dot(sr_ref[...] * inv, wl_ref[HH:H, :],
                            preferred_element_type=jnp.float32)
        acc = acc + jnp.dot(xl_ref[...], wr_ref[0:HH, :],
                            preferred_element_type=jnp.float32)
        acc = acc + jnp.dot(xr_ref[...], wr_ref[HH:H, :],
                            preferred_element_type=jnp.float32)
        acc = acc + b_ref[...]
        if relu:
            acc = jnp.maximum(acc, 0.0)
        if half_out:
            outs[0][...] = acc[:, 0:HH]
            outs[1][...] = acc[:, HH:H]
        else:
            outs[0][...] = acc

    if half_out:
        out_shape = (jax.ShapeDtypeStruct((N, HH), jnp.float32),
                     jax.ShapeDtypeStruct((N, HH), jnp.float32))
        out_specs = (pl.BlockSpec((BLK, HH), lambda i: (i, 0)),
                     pl.BlockSpec((BLK, HH), lambda i: (i, 0)))
    else:
        out_shape = jax.ShapeDtypeStruct((N, H), jnp.float32)
        out_specs = pl.BlockSpec((BLK, H), lambda i: (i, 0))

    return pl.pallas_call(
        body,
        grid=(N // BLK,),
        in_specs=[
            pl.BlockSpec((BLK, HH), lambda i: (i, 0)),
            pl.BlockSpec((BLK, HH), lambda i: (i, 0)),
            pl.BlockSpec((BLK, 1), lambda i: (i, 0)),
            pl.BlockSpec((BLK, HH), lambda i: (i, 0)),
            pl.BlockSpec((BLK, HH), lambda i: (i, 0)),
            pl.BlockSpec((H, H), lambda i: (0, 0)),
            pl.BlockSpec((1, H), lambda i: (0, 0)),
            pl.BlockSpec((H, H), lambda i: (0, 0)),
        ],
        out_specs=out_specs,
        out_shape=out_shape,
    )(sum_l, sum_r, cnt, x_l, x_r, w_l, b_l.reshape(1, H), w_r)


def kernel(ei_g2go, ei_go2g, gene_emb, go_emb,
           W1l_g2go, b1_g2go, W1r_g2go, W1l_go2g, b1_go2g, W1r_go2g,
           W2l_g2go, b2_g2go, W2r_g2go, W2l_go2g, b2_go2g, W2r_go2g):
    pad = jnp.arange(EPAD - E, dtype=jnp.int32)
    pad_src = pad % N
    pad_dst = N + pad % (NPAD - N)
    src_a = jnp.concatenate([ei_g2go[0], pad_src])
    dst_a = jnp.concatenate([ei_g2go[1], pad_dst])
    src_b = jnp.concatenate([ei_go2g[0], pad_src])
    dst_b = jnp.concatenate([ei_go2g[1], pad_dst])
    zeros_l = jnp.zeros((NPAD, HH), jnp.float32)
    gel, ger = gene_emb[:, :HH], gene_emb[:, HH:]
    gol, gor = go_emb[:, :HH], go_emb[:, HH:]

    s1go_l, s1go_r, hist_a = _agg_sc(gel, ger, src_a, dst_a, zeros_l, True)
    s1ge_l, s1ge_r, hist_b = _agg_sc(gol, gor, src_b, dst_b, zeros_l, True)
    cnt_go = jnp.sum(hist_a, axis=0).reshape(NPAD, 1)
    cnt_gene = jnp.sum(hist_b, axis=0).reshape(NPAD, 1)

    go1l, go1r = _dense_tc(s1go_l, s1go_r, cnt_go, gol, gor,
                           W1l_g2go, b1_g2go, W1r_g2go, True, True)
    ge1l, ge1r = _dense_tc(s1ge_l, s1ge_r, cnt_gene, gel, ger,
                           W1l_go2g, b1_go2g, W1r_go2g, True, True)

    s2go_l, s2go_r = _agg_sc(ge1l, ge1r, src_a, dst_a, zeros_l, False)
    s2ge_l, s2ge_r = _agg_sc(go1l, go1r, src_b, dst_b, zeros_l, False)

    go2 = _dense_tc(s2go_l, s2go_r, cnt_go, go1l, go1r,
                    W2l_g2go, b2_g2go, W2r_g2go, False, False)
    gene2 = _dense_tc(s2ge_l, s2ge_r, cnt_gene, ge1l, ge1r,
                      W2l_go2g, b2_go2g, W2r_go2g, False, False)
    return (gene2, go2)
